# baseline (speedup 1.0000x reference)
import functools

import jax
import jax.numpy as jnp
from jax.experimental import pallas as pl
from jax.experimental.pallas import tpu as pltpu

N = 10000
NPAD = 10240
D = 128
KP1 = 21
ROWS_BLK = 256
NCHUNK = NPAD // 128
TOPT = 6
EXACT_BLK = 400
NEG = float("-inf")


def _embed_kernel(x_ref, w1_ref, b1_ref, w2_ref, b2_ref, out_ref):
    x = x_ref[...]
    h = jax.lax.dot_general(x, w1_ref[...], (((1,), (1,)), ((), ())),
                            preferred_element_type=jnp.float32)
    h = jax.nn.relu(h + b1_ref[...])
    e = jax.lax.dot_general(h, w2_ref[...], (((1,), (1,)), ((), ())),
                            preferred_element_type=jnp.float32)
    e = e + b2_ref[...]
    nrm = jnp.sqrt(jnp.sum(e * e, axis=1, keepdims=True))
    nrm = jnp.maximum(nrm, 1e-12)
    out_ref[...] = e / nrm


def _topk_kernel(xn_ref, vals_ref, inds_ref, s_ref):
    i = pl.program_id(0)
    xb = xn_ref[pl.ds(i * ROWS_BLK, ROWS_BLK), :]
    sims = jax.lax.dot_general(xn_ref[...], xb, (((1,), (1,)), ((), ())),
                               preferred_element_type=jnp.float32)
    s_ref[...] = sims.reshape(NCHUNK, 128, ROWS_BLK)
    npc = N // 128
    plane = jax.lax.broadcasted_iota(
        jnp.int32, (NCHUNK - npc, 128, ROWS_BLK), 1)
    pchunk = jax.lax.broadcasted_iota(
        jnp.int32, (NCHUNK - npc, 128, ROWS_BLK), 0) + npc
    s_ref[npc:, :, :] = jnp.where(pchunk * 128 + plane < N,
                                  s_ref[npc:, :, :], NEG)

    cix = jax.lax.broadcasted_iota(jnp.int32, (NCHUNK, 128, ROWS_BLK), 0)
    lane = jax.lax.broadcasted_iota(jnp.int32, (128, ROWS_BLK), 0)

    mvals = []
    mchunks = []
    dead = None
    for t in range(TOPT):
        s3 = s_ref[...]
        masked = s3 if dead is None else jnp.where(dead, NEG, s3)
        mv = jnp.max(masked, axis=0)
        mc = jnp.min(jnp.where(masked == mv[None], cix, NCHUNK), axis=0)
        hit = cix == mc[None]
        dead = hit if dead is None else (dead | hit)
        mvals.append(mv)
        mchunks.append(mc)

    pool_v = jnp.concatenate(mvals, axis=0)
    pool_i = jnp.concatenate(
        [mc * 128 + lane for mc in mchunks], axis=0).astype(jnp.int32)
    pos = jax.lax.broadcasted_iota(jnp.int32, (128 * TOPT, ROWS_BLK), 0)
    last_slot = 128 * (TOPT - 1)

    vs = []
    ids = []
    exhausted = jnp.zeros((1, ROWS_BLK), jnp.bool_)
    pv = pool_v
    for it in range(KP1):
        v = jnp.max(pv, axis=0, keepdims=True)
        idx = jnp.min(jnp.where(pv == v, pool_i, NPAD * 2), axis=0,
                      keepdims=True)
        hit = (pv == v) & (pool_i == idx)
        if it < KP1 - 1:
            drained = jnp.max(jnp.where(hit, pos, -1), axis=0,
                              keepdims=True) >= last_slot
            exhausted = exhausted | drained
        vs.append(v)
        ids.append(idx)
        pv = jnp.where(hit, NEG, pv)
    vals_ref[...] = jnp.concatenate(vs, axis=0)
    inds_ref[...] = jnp.concatenate(
        ids + [exhausted.astype(jnp.int32)], axis=0)


def _exact_topk_kernel(xn_ref, vals_ref, inds_ref):
    i = pl.program_id(0)
    xb = xn_ref[pl.ds(i * EXACT_BLK, EXACT_BLK), :]
    sims = jax.lax.dot_general(xb, xn_ref[...], (((1,), (1,)), ((), ())),
                               preferred_element_type=jnp.float32)
    col = jax.lax.broadcasted_iota(jnp.int32, (EXACT_BLK, N), 1)
    vs = []
    ids = []
    s = sims
    for _ in range(KP1):
        v = jnp.max(s, axis=1)
        idx = jnp.min(jnp.where(s == v[:, None], col, N), axis=1)
        vs.append(v)
        ids.append(idx)
        s = jnp.where(col == idx[:, None], NEG, s)
    vals_ref[...] = jnp.stack(vs, axis=1)
    inds_ref[...] = jnp.stack(ids, axis=1)


@functools.partial(jax.jit, static_argnames=())
def kernel(features, W1, b1, W2, b2):
    xn = pl.pallas_call(
        _embed_kernel,
        grid=(10,),
        in_specs=[
            pl.BlockSpec((N // 10, D), lambda i: (i, 0)),
            pl.BlockSpec((D, D), lambda i: (0, 0)),
            pl.BlockSpec((1, D), lambda i: (0, 0)),
            pl.BlockSpec((D, D), lambda i: (0, 0)),
            pl.BlockSpec((1, D), lambda i: (0, 0)),
        ],
        out_specs=pl.BlockSpec((N // 10, D), lambda i: (i, 0)),
        out_shape=jax.ShapeDtypeStruct((N, D), jnp.float32),
    )(features, W1, b1.reshape(1, D), W2, b2.reshape(1, D))

    xn_pad = jnp.zeros((NPAD, D), jnp.float32).at[:N].set(xn)

    vals_t, inds_t = pl.pallas_call(
        _topk_kernel,
        grid=(NPAD // ROWS_BLK,),
        in_specs=[pl.BlockSpec((NPAD, D), lambda i: (0, 0))],
        out_specs=[
            pl.BlockSpec((KP1, ROWS_BLK), lambda i: (0, i)),
            pl.BlockSpec((KP1 + 1, ROWS_BLK), lambda i: (0, i)),
        ],
        out_shape=[
            jax.ShapeDtypeStruct((KP1, NPAD), jnp.float32),
            jax.ShapeDtypeStruct((KP1 + 1, NPAD), jnp.int32),
        ],
        scratch_shapes=[pltpu.VMEM((NCHUNK, 128, ROWS_BLK), jnp.float32)],
    )(xn_pad)

    need_fb = jnp.any(inds_t[KP1] != 0)

    def _exact_path(xnp):
        return pl.pallas_call(
            _exact_topk_kernel,
            grid=(N // EXACT_BLK,),
            in_specs=[pl.BlockSpec((N, D), lambda i: (0, 0))],
            out_specs=[
                pl.BlockSpec((EXACT_BLK, KP1), lambda i: (i, 0)),
                pl.BlockSpec((EXACT_BLK, KP1), lambda i: (i, 0)),
            ],
            out_shape=[
                jax.ShapeDtypeStruct((N, KP1), jnp.float32),
                jax.ShapeDtypeStruct((N, KP1), jnp.int32),
            ],
        )(xnp[:N])

    def _pooled_path(_):
        return vals_t[:, :N].T, inds_t[:KP1, :N].T

    vals, inds = jax.lax.cond(need_fb, _exact_path, _pooled_path, xn_pad)

    values = vals.reshape(-1)
    cols = inds.reshape(-1)
    rows = jnp.repeat(jnp.arange(N, dtype=jnp.int32), KP1)
    edge_index = jnp.stack([jnp.concatenate([rows, cols]),
                            jnp.concatenate([cols, rows])])
    edge_weight = jax.nn.relu(jnp.concatenate([values, values]))
    return edge_index, edge_weight


# speedup vs baseline: 1.1178x; 1.1178x over previous
import functools

import jax
import jax.numpy as jnp
from jax.experimental import pallas as pl
from jax.experimental.pallas import tpu as pltpu

N = 10000
NPAD = 10240
D = 128
KP1 = 21
ROWS_BLK = 256
NCHUNK = NPAD // 128
TOPT = 6
EXACT_BLK = 400
NEG = float("-inf")


def _embed_kernel(x_ref, w1_ref, b1_ref, w2_ref, b2_ref, out_ref):
    x = x_ref[...]
    h = jax.lax.dot_general(x, w1_ref[...], (((1,), (1,)), ((), ())),
                            preferred_element_type=jnp.float32)
    h = jax.nn.relu(h + b1_ref[...])
    e = jax.lax.dot_general(h, w2_ref[...], (((1,), (1,)), ((), ())),
                            preferred_element_type=jnp.float32)
    e = e + b2_ref[...]
    nrm = jnp.sqrt(jnp.sum(e * e, axis=1, keepdims=True))
    nrm = jnp.maximum(nrm, 1e-12)
    out_ref[...] = e / nrm


def _topk_kernel(xn_ref, vals_ref, inds_ref, s_ref):
    i = pl.program_id(0)
    xb = xn_ref[pl.ds(i * ROWS_BLK, ROWS_BLK), :]
    sims = jax.lax.dot_general(xn_ref[...], xb, (((1,), (1,)), ((), ())),
                               preferred_element_type=jnp.float32)
    s_ref[...] = sims.reshape(NCHUNK, 128, ROWS_BLK)
    npc = N // 128
    plane = jax.lax.broadcasted_iota(
        jnp.int32, (NCHUNK - npc, 128, ROWS_BLK), 1)
    pchunk = jax.lax.broadcasted_iota(
        jnp.int32, (NCHUNK - npc, 128, ROWS_BLK), 0) + npc
    s_ref[npc:, :, :] = jnp.where(pchunk * 128 + plane < N,
                                  s_ref[npc:, :, :], NEG)

    cix = jax.lax.broadcasted_iota(jnp.int32, (NCHUNK, 128, ROWS_BLK), 0)
    lane = jax.lax.broadcasted_iota(jnp.int32, (128, ROWS_BLK), 0)

    mvals = []
    mchunks = []
    dead = None
    for t in range(TOPT):
        s3 = s_ref[...]
        masked = s3 if dead is None else jnp.where(dead, NEG, s3)
        mv = jnp.max(masked, axis=0)
        mc = jnp.min(jnp.where(masked == mv[None], cix, NCHUNK), axis=0)
        hit = cix == mc[None]
        dead = hit if dead is None else (dead | hit)
        mvals.append(mv)
        mchunks.append(mc)

    pool_v = jnp.concatenate(mvals, axis=0)
    pool_i = jnp.concatenate(
        [mc * 128 + lane for mc in mchunks], axis=0).astype(jnp.int32)
    vs = []
    ids = []
    pv = pool_v
    for it in range(KP1):
        v = jnp.max(pv, axis=0, keepdims=True)
        idx = jnp.min(jnp.where(pv == v, pool_i, NPAD * 2), axis=0,
                      keepdims=True)
        hit = (pv == v) & (pool_i == idx)
        vs.append(v)
        ids.append(idx)
        pv = jnp.where(hit, NEG, pv)
    popped = (pv == NEG).astype(jnp.int32).reshape(TOPT, 128, ROWS_BLK)
    cnt = jnp.sum(popped, axis=0)
    exhausted = jnp.max((cnt >= TOPT).astype(jnp.int32), axis=0,
                        keepdims=True) > 0
    vals_ref[...] = jnp.concatenate(vs, axis=0)
    inds_ref[...] = jnp.concatenate(
        ids + [exhausted.astype(jnp.int32)], axis=0)


def _exact_topk_kernel(xn_ref, vals_ref, inds_ref):
    i = pl.program_id(0)
    xb = xn_ref[pl.ds(i * EXACT_BLK, EXACT_BLK), :]
    sims = jax.lax.dot_general(xb, xn_ref[...], (((1,), (1,)), ((), ())),
                               preferred_element_type=jnp.float32)
    col = jax.lax.broadcasted_iota(jnp.int32, (EXACT_BLK, N), 1)
    vs = []
    ids = []
    s = sims
    for _ in range(KP1):
        v = jnp.max(s, axis=1)
        idx = jnp.min(jnp.where(s == v[:, None], col, N), axis=1)
        vs.append(v)
        ids.append(idx)
        s = jnp.where(col == idx[:, None], NEG, s)
    vals_ref[...] = jnp.stack(vs, axis=1)
    inds_ref[...] = jnp.stack(ids, axis=1)


@functools.partial(jax.jit, static_argnames=())
def kernel(features, W1, b1, W2, b2):
    xn = pl.pallas_call(
        _embed_kernel,
        grid=(10,),
        in_specs=[
            pl.BlockSpec((N // 10, D), lambda i: (i, 0)),
            pl.BlockSpec((D, D), lambda i: (0, 0)),
            pl.BlockSpec((1, D), lambda i: (0, 0)),
            pl.BlockSpec((D, D), lambda i: (0, 0)),
            pl.BlockSpec((1, D), lambda i: (0, 0)),
        ],
        out_specs=pl.BlockSpec((N // 10, D), lambda i: (i, 0)),
        out_shape=jax.ShapeDtypeStruct((N, D), jnp.float32),
    )(features, W1, b1.reshape(1, D), W2, b2.reshape(1, D))

    xn_pad = jnp.zeros((NPAD, D), jnp.float32).at[:N].set(xn)

    vals_t, inds_t = pl.pallas_call(
        _topk_kernel,
        grid=(NPAD // ROWS_BLK,),
        in_specs=[pl.BlockSpec((NPAD, D), lambda i: (0, 0))],
        out_specs=[
            pl.BlockSpec((KP1, ROWS_BLK), lambda i: (0, i)),
            pl.BlockSpec((KP1 + 1, ROWS_BLK), lambda i: (0, i)),
        ],
        out_shape=[
            jax.ShapeDtypeStruct((KP1, NPAD), jnp.float32),
            jax.ShapeDtypeStruct((KP1 + 1, NPAD), jnp.int32),
        ],
        scratch_shapes=[pltpu.VMEM((NCHUNK, 128, ROWS_BLK), jnp.float32)],
    )(xn_pad)

    need_fb = jnp.any(inds_t[KP1] != 0)

    def _exact_path(xnp):
        return pl.pallas_call(
            _exact_topk_kernel,
            grid=(N // EXACT_BLK,),
            in_specs=[pl.BlockSpec((N, D), lambda i: (0, 0))],
            out_specs=[
                pl.BlockSpec((EXACT_BLK, KP1), lambda i: (i, 0)),
                pl.BlockSpec((EXACT_BLK, KP1), lambda i: (i, 0)),
            ],
            out_shape=[
                jax.ShapeDtypeStruct((N, KP1), jnp.float32),
                jax.ShapeDtypeStruct((N, KP1), jnp.int32),
            ],
        )(xnp[:N])

    def _pooled_path(_):
        return vals_t[:, :N].T, inds_t[:KP1, :N].T

    vals, inds = jax.lax.cond(need_fb, _exact_path, _pooled_path, xn_pad)

    values = vals.reshape(-1)
    cols = inds.reshape(-1)
    rows = jnp.repeat(jnp.arange(N, dtype=jnp.int32), KP1)
    edge_index = jnp.stack([jnp.concatenate([rows, cols]),
                            jnp.concatenate([cols, rows])])
    edge_weight = jax.nn.relu(jnp.concatenate([values, values]))
    return edge_index, edge_weight
